# baseline (speedup 1.0000x reference)
import functools

import jax
import jax.numpy as jnp
from jax import lax
from jax.experimental import pallas as pl
from jax.experimental.pallas import tpu as pltpu
from jax.experimental.pallas import tpu_sc as plsc

B = 16384
D = 64
NC = 2
NS = 16
NW = NC * NS
BPW = B // NW
CHUNK = 128
NCHUNK = BPW // CHUNK
GROUPS = BPW // 16


def _fm_kernel(user_h, item_h, uw_h, iw_h, ub_h, ib_h, bias_h, out_h,
               uidx_v, iidx_v, urow_v, irow_v, ub_v, ib_v, bias_v, out_v,
               sem):
    wid = lax.axis_index("s") * NC + lax.axis_index("c")

    pltpu.sync_copy(user_h.at[wid], uidx_v)
    pltpu.sync_copy(item_h.at[wid], iidx_v)
    pltpu.sync_copy(bias_h, bias_v)

    for j in range(NCHUNK):
        pltpu.async_copy(uw_h.at[uidx_v.at[j]],
                         urow_v.at[pl.ds(j * CHUNK, CHUNK)], sem)
        pltpu.async_copy(iw_h.at[iidx_v.at[j]],
                         irow_v.at[pl.ds(j * CHUNK, CHUNK)], sem)
        pltpu.async_copy(ub_h.at[uidx_v.at[j]],
                         ub_v.at[pl.ds(j * CHUNK, CHUNK)], sem)
        pltpu.async_copy(ib_h.at[iidx_v.at[j]],
                         ib_v.at[pl.ds(j * CHUNK, CHUNK)], sem)
    for j in range(NCHUNK):
        pltpu.make_async_copy(uw_h.at[uidx_v.at[j]],
                              urow_v.at[pl.ds(j * CHUNK, CHUNK)], sem).wait()
        pltpu.make_async_copy(iw_h.at[iidx_v.at[j]],
                              irow_v.at[pl.ds(j * CHUNK, CHUNK)], sem).wait()
        pltpu.make_async_copy(ub_h.at[uidx_v.at[j]],
                              ub_v.at[pl.ds(j * CHUNK, CHUNK)], sem).wait()
        pltpu.make_async_copy(ib_h.at[iidx_v.at[j]],
                              ib_v.at[pl.ds(j * CHUNK, CHUNK)], sem).wait()

    b0 = bias_v[...]

    def group_body(g, carry):
        rows = g * 16 + lax.iota(jnp.int32, 16)
        acc = ub_v[pl.ds(g * 16, 16)] + ib_v[pl.ds(g * 16, 16)] + b0
        for f in range(D):
            cols = jnp.full((16,), f, jnp.int32)
            gu = plsc.load_gather(urow_v, [rows, cols])
            gi = plsc.load_gather(irow_v, [rows, cols])
            acc = acc + gu * gi
        out_v[pl.ds(g * 16, 16)] = acc
        return carry

    lax.fori_loop(0, GROUPS, group_body, 0)
    pltpu.sync_copy(out_v, out_h.at[pl.ds(wid * BPW, BPW)])


def kernel(user, item, context, embed_user_w, embed_item_w,
           u_bias_w, i_bias_w, bias_):
    del context
    user3 = user.astype(jnp.int32).reshape(NW, NCHUNK, CHUNK)
    item3 = item.astype(jnp.int32).reshape(NW, NCHUNK, CHUNK)
    ub_flat = u_bias_w.reshape(-1)
    ib_flat = i_bias_w.reshape(-1)
    bias16 = jnp.broadcast_to(bias_.reshape(()), (16,))

    mesh = plsc.VectorSubcoreMesh(core_axis_name="c", subcore_axis_name="s")
    fm = functools.partial(
        pl.kernel,
        out_type=jax.ShapeDtypeStruct((B,), jnp.float32),
        mesh=mesh,
        compiler_params=pltpu.CompilerParams(
            needs_layout_passes=False, use_tc_tiling_on_sc=False),
        scratch_types=[
            pltpu.VMEM((NCHUNK, CHUNK), jnp.int32),
            pltpu.VMEM((NCHUNK, CHUNK), jnp.int32),
            pltpu.VMEM((BPW, D), jnp.float32),
            pltpu.VMEM((BPW, D), jnp.float32),
            pltpu.VMEM((BPW,), jnp.float32),
            pltpu.VMEM((BPW,), jnp.float32),
            pltpu.VMEM((16,), jnp.float32),
            pltpu.VMEM((BPW,), jnp.float32),
            pltpu.SemaphoreType.DMA,
        ],
    )(_fm_kernel)
    return fm(user3, item3, embed_user_w, embed_item_w, ub_flat, ib_flat,
              bias16)
